# baseline (speedup 1.0000x reference)
import functools

import jax
import jax.numpy as jnp
from jax import lax
from jax.experimental import pallas as pl
from jax.experimental.pallas import tpu as pltpu
from jax.experimental.pallas import tpu_sc as plsc

M = 64
CH = 256
AUG = 128


def _router_body(x_ref, wr_ref, inv_ref, xa_ref, be_ref, bm_ref):
    n, d = x_ref.shape
    e = wr_ref.shape[0]
    nb = be_ref.shape[0]
    x = x_ref[...]
    wr = wr_ref[...]
    logits = lax.dot_general(x, wr, (((1,), (1,)), ((), ())),
                             preferred_element_type=jnp.float32)
    m = jnp.max(logits, axis=1, keepdims=True)
    s = jnp.sum(jnp.exp(logits - m), axis=1)
    top = 1.0 / s
    cw = top / (top + 1e-8)
    xa_ref[:, :d] = x
    xa_ref[:, d:] = jnp.broadcast_to(cw[:, None], (n, AUG))
    iota_e = lax.broadcasted_iota(jnp.int32, (n, e), 1)
    eid = jnp.min(jnp.where(logits >= m, iota_e, e), axis=1)
    oh = (iota_e == eid[:, None]).astype(jnp.float32)
    ir = lax.broadcasted_iota(jnp.int32, (CH, CH), 0)
    jr = lax.broadcasted_iota(jnp.int32, (CH, CH), 1)
    ls = (jr < ir).astype(jnp.float32)
    rank_rows = []
    carry = jnp.zeros((1, e), jnp.float32)
    for c in range(n // CH):
        oh_c = oh[c * CH:(c + 1) * CH, :]
        r_c = lax.dot_general(ls, oh_c, (((1,), (0,)), ((), ())),
                              preferred_element_type=jnp.float32)
        rank_rows.append(r_c + carry)
        carry = carry + jnp.sum(oh_c, axis=0, keepdims=True)
    rank = jnp.concatenate(rank_rows, axis=0)
    counts = carry
    pcb = jnp.floor((counts + (M - 1)) / M)
    iu = lax.broadcasted_iota(jnp.int32, (e, e), 0)
    ju = lax.broadcasted_iota(jnp.int32, (e, e), 1)
    ut = (iu <= ju).astype(jnp.float32)
    cumb = lax.dot_general(pcb, ut, (((1,), (0,)), ((), ())),
                           preferred_element_type=jnp.float32)
    poff = (cumb - pcb) * M
    pos = jnp.sum(oh * (rank + poff), axis=1)
    inv_ref[...] = pos.astype(jnp.int32)
    used = cumb[:, e - 1:e]
    bi = lax.broadcasted_iota(jnp.int32, (nb, e), 0).astype(jnp.float32)
    bi = jnp.minimum(bi, used - 1.0)
    cnt = jnp.sum((cumb <= bi).astype(jnp.int32), axis=1)
    be_ref[...] = jnp.minimum(cnt, e - 1).astype(jnp.int32)
    bm_ref[...] = bi[:, 0].astype(jnp.int32)


def _router(xf, Wr, nb):
    n, d = xf.shape
    return pl.pallas_call(
        _router_body,
        out_shape=(jax.ShapeDtypeStruct((n,), jnp.int32),
                   jax.ShapeDtypeStruct((n, d + AUG), jnp.float32),
                   jax.ShapeDtypeStruct((nb,), jnp.int32),
                   jax.ShapeDtypeStruct((nb,), jnp.int32)),
    )(xf, Wr)


def _sc_gather(table, idx, chunk):
    v, d = table.shape
    b = idx.shape[0]
    info = plsc.get_sparse_core_info()
    nc = info.num_cores
    nw = nc * info.num_subcores
    b_per_w = b // nw
    nchunks = b_per_w // chunk
    mesh = plsc.VectorSubcoreMesh(core_axis_name="c", subcore_axis_name="s")

    @functools.partial(
        pl.kernel, mesh=mesh,
        out_type=jax.ShapeDtypeStruct((b, d), table.dtype),
        scratch_types=[
            pltpu.VMEM((chunk,), jnp.int32),
            pltpu.VMEM((chunk, d), table.dtype),
            pltpu.SemaphoreType.DMA,
        ],
    )
    def k(table_hbm, idx_hbm, out_hbm, idx_v, rows_v, sem):
        wid = lax.axis_index("s") * nc + lax.axis_index("c")
        base = wid * b_per_w
        for c in range(nchunks):
            o = base + c * chunk
            pltpu.sync_copy(idx_hbm.at[pl.ds(o, chunk)], idx_v)
            pltpu.async_copy(table_hbm.at[idx_v], rows_v, sem).wait()
            pltpu.sync_copy(rows_v, out_hbm.at[pl.ds(o, chunk)])

    return k(table, idx)


def _sc_scatter(rows, idx, npad, chunk):
    n, d = rows.shape
    info = plsc.get_sparse_core_info()
    nc = info.num_cores
    nw = nc * info.num_subcores
    n_per_w = n // nw
    nchunks = n_per_w // chunk
    mesh = plsc.VectorSubcoreMesh(core_axis_name="c", subcore_axis_name="s")

    @functools.partial(
        pl.kernel, mesh=mesh,
        out_type=jax.ShapeDtypeStruct((npad, d), rows.dtype),
        scratch_types=[
            pltpu.VMEM((chunk,), jnp.int32),
            pltpu.VMEM((chunk, d), rows.dtype),
            pltpu.SemaphoreType.DMA,
        ],
    )
    def k(rows_hbm, idx_hbm, out_hbm, idx_v, rows_v, sem):
        wid = lax.axis_index("s") * nc + lax.axis_index("c")
        base = wid * n_per_w
        for c in range(nchunks):
            o = base + c * chunk
            pltpu.sync_copy(idx_hbm.at[pl.ds(o, chunk)], idx_v)
            pltpu.sync_copy(rows_hbm.at[pl.ds(o, chunk)], rows_v)
            pltpu.async_copy(rows_v, out_hbm.at[idx_v], sem).wait()

    return k(rows, idx)


def _ffn_body(s_ref, xs_ref, wg_ref, wu_ref, wd_ref,
              wgs_ref, wus_ref, wds_ref, out_ref):
    d = out_ref.shape[1]
    xa = xs_ref[...]
    xb = xa[:, :d].astype(jnp.bfloat16)
    cw = xa[:, d:d + 1]
    wg = wg_ref[0].astype(jnp.bfloat16)
    wu = wu_ref[0].astype(jnp.bfloat16)
    wd = wd_ref[0].astype(jnp.bfloat16)
    g = lax.dot_general(xb, wg, (((1,), (1,)), ((), ())),
                        preferred_element_type=jnp.float32)
    u = lax.dot_general(xb, wu, (((1,), (1,)), ((), ())),
                        preferred_element_type=jnp.float32)
    h = (g * jax.nn.sigmoid(g) * u * cw).astype(jnp.bfloat16)
    o = lax.dot_general(h, wd, (((1,), (1,)), ((), ())),
                        preferred_element_type=jnp.float32)
    wgs = wgs_ref[...].astype(jnp.bfloat16)
    wus = wus_ref[...].astype(jnp.bfloat16)
    wds = wds_ref[...].astype(jnp.bfloat16)
    gs = lax.dot_general(xb, wgs, (((1,), (1,)), ((), ())),
                         preferred_element_type=jnp.float32)
    us = lax.dot_general(xb, wus, (((1,), (1,)), ((), ())),
                         preferred_element_type=jnp.float32)
    hs = (gs * jax.nn.sigmoid(gs) * us).astype(jnp.bfloat16)
    out_ref[...] = o + lax.dot_general(hs, wds, (((1,), (1,)), ((), ())),
                                       preferred_element_type=jnp.float32)


def _ffn(bebm, xs, Wg, Wu, Wd, Wgs, Wus, Wds):
    nb = bebm.shape[1]
    npad, da = xs.shape
    e, f, d = Wg.shape
    grid_spec = pltpu.PrefetchScalarGridSpec(
        num_scalar_prefetch=1,
        grid=(nb,),
        in_specs=[
            pl.BlockSpec((M, da), lambda b, s: (s[1, b], 0)),
            pl.BlockSpec((1, f, d), lambda b, s: (s[0, b], 0, 0)),
            pl.BlockSpec((1, f, d), lambda b, s: (s[0, b], 0, 0)),
            pl.BlockSpec((1, d, f), lambda b, s: (s[0, b], 0, 0)),
            pl.BlockSpec((f, d), lambda b, s: (0, 0)),
            pl.BlockSpec((f, d), lambda b, s: (0, 0)),
            pl.BlockSpec((d, f), lambda b, s: (0, 0)),
        ],
        out_specs=pl.BlockSpec((M, d), lambda b, s: (s[1, b], 0)),
    )
    return pl.pallas_call(
        _ffn_body,
        grid_spec=grid_spec,
        out_shape=jax.ShapeDtypeStruct((npad, d), jnp.float32),
    )(bebm, xs, Wg, Wu, Wd, Wgs, Wus, Wds)


def kernel(x, Wr, Wg, Wu, Wd, Wgs, Wus, Wds):
    b, t, d = x.shape
    n = b * t
    e, f, _ = Wg.shape
    nb = n // M + e
    npad = nb * M
    xf = x.reshape(n, d)
    inv, xa, be, bm = _router(xf, Wr, nb)
    xs = _sc_scatter(xa, inv, npad, 64)
    outs = _ffn(jnp.stack([be, bm]), xs, Wg, Wu, Wd, Wgs, Wus, Wds)
    out = _sc_gather(outs, inv, 64)
    return out.reshape(b, t, d)


# speedup vs baseline: 1.1042x; 1.1042x over previous
import functools

import jax
import jax.numpy as jnp
from jax import lax
from jax.experimental import pallas as pl
from jax.experimental.pallas import tpu as pltpu
from jax.experimental.pallas import tpu_sc as plsc

M = 64
CH = 256


def _router_body(x_ref, wr_ref, inv_ref, cw_ref, be_ref, bm_ref):
    n, d = x_ref.shape
    e = wr_ref.shape[0]
    nb = be_ref.shape[0]
    x = x_ref[...]
    wr = wr_ref[...]
    logits = lax.dot_general(x, wr, (((1,), (1,)), ((), ())),
                             preferred_element_type=jnp.float32)
    m = jnp.max(logits, axis=1, keepdims=True)
    s = jnp.sum(jnp.exp(logits - m), axis=1)
    top = 1.0 / s
    cw_ref[...] = top / (top + 1e-8)
    iota_e = lax.broadcasted_iota(jnp.int32, (n, e), 1)
    eid = jnp.min(jnp.where(logits >= m, iota_e, e), axis=1)
    oh = (iota_e == eid[:, None]).astype(jnp.float32)
    ir = lax.broadcasted_iota(jnp.int32, (CH, CH), 0)
    jr = lax.broadcasted_iota(jnp.int32, (CH, CH), 1)
    ls = (jr < ir).astype(jnp.float32)
    rank_rows = []
    carry = jnp.zeros((1, e), jnp.float32)
    for c in range(n // CH):
        oh_c = oh[c * CH:(c + 1) * CH, :]
        r_c = lax.dot_general(ls, oh_c, (((1,), (0,)), ((), ())),
                              preferred_element_type=jnp.float32)
        rank_rows.append(r_c + carry)
        carry = carry + jnp.sum(oh_c, axis=0, keepdims=True)
    rank = jnp.concatenate(rank_rows, axis=0)
    counts = carry
    pcb = jnp.floor((counts + (M - 1)) / M)
    iu = lax.broadcasted_iota(jnp.int32, (e, e), 0)
    ju = lax.broadcasted_iota(jnp.int32, (e, e), 1)
    ut = (iu <= ju).astype(jnp.float32)
    cumb = lax.dot_general(pcb, ut, (((1,), (0,)), ((), ())),
                           preferred_element_type=jnp.float32)
    poff = (cumb - pcb) * M
    pos = jnp.sum(oh * (rank + poff), axis=1)
    inv_ref[...] = pos.astype(jnp.int32)
    used = cumb[:, e - 1:e]
    bi = lax.broadcasted_iota(jnp.int32, (nb, e), 0).astype(jnp.float32)
    bi = jnp.minimum(bi, used - 1.0)
    cnt = jnp.sum((cumb <= bi).astype(jnp.int32), axis=1)
    be_ref[...] = jnp.minimum(cnt, e - 1).astype(jnp.int32)
    bm_ref[...] = bi[:, 0].astype(jnp.int32)


def _router(xf, Wr, nb):
    n = xf.shape[0]
    return pl.pallas_call(
        _router_body,
        out_shape=(jax.ShapeDtypeStruct((n,), jnp.int32),
                   jax.ShapeDtypeStruct((n,), jnp.float32),
                   jax.ShapeDtypeStruct((nb,), jnp.int32),
                   jax.ShapeDtypeStruct((nb,), jnp.int32)),
    )(xf, Wr)


def _sc_gather(table, idx, chunk):
    v, d = table.shape
    b = idx.shape[0]
    info = plsc.get_sparse_core_info()
    nc = info.num_cores
    nw = nc * info.num_subcores
    b_per_w = b // nw
    nchunks = b_per_w // chunk
    mesh = plsc.VectorSubcoreMesh(core_axis_name="c", subcore_axis_name="s")

    @functools.partial(
        pl.kernel, mesh=mesh,
        out_type=jax.ShapeDtypeStruct((b, d), jnp.float32),
        scratch_types=[
            pltpu.VMEM((chunk,), jnp.int32),
            pltpu.VMEM((chunk, d), jnp.float32),
            pltpu.SemaphoreType.DMA,
        ],
    )
    def k(table_hbm, idx_hbm, out_hbm, idx_v, rows_v, sem):
        wid = lax.axis_index("s") * nc + lax.axis_index("c")
        base = wid * b_per_w
        for c in range(nchunks):
            o = base + c * chunk
            pltpu.sync_copy(idx_hbm.at[pl.ds(o, chunk)], idx_v)
            pltpu.async_copy(table_hbm.at[idx_v], rows_v, sem).wait()
            pltpu.sync_copy(rows_v, out_hbm.at[pl.ds(o, chunk)])

    return k(table, idx)


def _sc_scatter(rows, idx, npad, chunk):
    n, d = rows.shape
    info = plsc.get_sparse_core_info()
    nc = info.num_cores
    nw = nc * info.num_subcores
    n_per_w = n // nw
    nchunks = n_per_w // chunk
    mesh = plsc.VectorSubcoreMesh(core_axis_name="c", subcore_axis_name="s")

    @functools.partial(
        pl.kernel, mesh=mesh,
        out_type=jax.ShapeDtypeStruct((npad, d), jnp.float32),
        scratch_types=[
            pltpu.VMEM((chunk,), jnp.int32),
            pltpu.VMEM((chunk, d), jnp.float32),
            pltpu.SemaphoreType.DMA,
        ],
    )
    def k(rows_hbm, idx_hbm, out_hbm, idx_v, rows_v, sem):
        wid = lax.axis_index("s") * nc + lax.axis_index("c")
        base = wid * n_per_w
        for c in range(nchunks):
            o = base + c * chunk
            pltpu.sync_copy(idx_hbm.at[pl.ds(o, chunk)], idx_v)
            pltpu.sync_copy(rows_hbm.at[pl.ds(o, chunk)], rows_v)
            pltpu.async_copy(rows_v, out_hbm.at[idx_v], sem).wait()

    return k(rows, idx)


def _ffn_body(be_ref, xs_ref, wg_ref, wu_ref, wd_ref, out_ref):
    xb = xs_ref[...].astype(jnp.bfloat16)
    wg = wg_ref[0].astype(jnp.bfloat16)
    wu = wu_ref[0].astype(jnp.bfloat16)
    wd = wd_ref[0].astype(jnp.bfloat16)
    g = lax.dot_general(xb, wg, (((1,), (1,)), ((), ())),
                        preferred_element_type=jnp.float32)
    u = lax.dot_general(xb, wu, (((1,), (1,)), ((), ())),
                        preferred_element_type=jnp.float32)
    h = (g * jax.nn.sigmoid(g) * u).astype(jnp.bfloat16)
    out_ref[...] = lax.dot_general(h, wd, (((1,), (1,)), ((), ())),
                                   preferred_element_type=jnp.float32)


def _ffn(bebm, xs, Wg, Wu, Wd):
    nb = bebm.shape[1]
    npad, d = xs.shape
    e, f, _ = Wg.shape
    grid_spec = pltpu.PrefetchScalarGridSpec(
        num_scalar_prefetch=1,
        grid=(nb,),
        in_specs=[
            pl.BlockSpec((M, d), lambda b, s: (s[1, b], 0)),
            pl.BlockSpec((1, f, d), lambda b, s: (s[0, b], 0, 0)),
            pl.BlockSpec((1, f, d), lambda b, s: (s[0, b], 0, 0)),
            pl.BlockSpec((1, d, f), lambda b, s: (s[0, b], 0, 0)),
        ],
        out_specs=pl.BlockSpec((M, d), lambda b, s: (s[1, b], 0)),
    )
    return pl.pallas_call(
        _ffn_body,
        grid_spec=grid_spec,
        out_shape=jax.ShapeDtypeStruct((npad, d), jnp.float32),
    )(bebm, xs, Wg, Wu, Wd)


def _shared_body(x_ref, moe_ref, cw_ref, wgs_ref, wus_ref, wds_ref, out_ref):
    xb = x_ref[...].astype(jnp.bfloat16)
    wgs = wgs_ref[...].astype(jnp.bfloat16)
    wus = wus_ref[...].astype(jnp.bfloat16)
    wds = wds_ref[...].astype(jnp.bfloat16)
    g = lax.dot_general(xb, wgs, (((1,), (1,)), ((), ())),
                        preferred_element_type=jnp.float32)
    u = lax.dot_general(xb, wus, (((1,), (1,)), ((), ())),
                        preferred_element_type=jnp.float32)
    h = (g * jax.nn.sigmoid(g) * u).astype(jnp.bfloat16)
    o = lax.dot_general(h, wds, (((1,), (1,)), ((), ())),
                        preferred_element_type=jnp.float32)
    out_ref[...] = moe_ref[...] * cw_ref[0, 0, :][:, None] + o


def _shared(xf, moe, cw3, Wgs, Wus, Wds):
    n, d = xf.shape
    f = Wgs.shape[0]
    ms = 256
    return pl.pallas_call(
        _shared_body,
        grid=(n // ms,),
        in_specs=[
            pl.BlockSpec((ms, d), lambda i: (i, 0)),
            pl.BlockSpec((ms, d), lambda i: (i, 0)),
            pl.BlockSpec((1, 1, ms), lambda i: (i, 0, 0)),
            pl.BlockSpec((f, d), lambda i: (0, 0)),
            pl.BlockSpec((f, d), lambda i: (0, 0)),
            pl.BlockSpec((d, f), lambda i: (0, 0)),
        ],
        out_specs=pl.BlockSpec((ms, d), lambda i: (i, 0)),
        out_shape=jax.ShapeDtypeStruct((n, d), jnp.float32),
    )(xf, moe, cw3, Wgs, Wus, Wds)


def kernel(x, Wr, Wg, Wu, Wd, Wgs, Wus, Wds):
    b, t, d = x.shape
    n = b * t
    e, f, _ = Wg.shape
    nb = n // M + e
    npad = nb * M
    xf = x.reshape(n, d)
    inv, cw, be, bm = _router(xf, Wr, nb)
    xs = _sc_scatter(xf, inv, npad, 64)
    outs = _ffn(jnp.stack([be, bm]), xs, Wg, Wu, Wd)
    moe = _sc_gather(outs, inv, 64)
    cw3 = cw.reshape(n // 256, 1, 256)
    out = _shared(xf, moe, cw3, Wgs, Wus, Wds)
    return out.reshape(b, t, d)
